# baseline (speedup 1.0000x reference)
import functools

import jax
import jax.numpy as jnp
from jax import lax
from jax.experimental import pallas as pl
from jax.experimental.pallas import tpu as pltpu
from jax.experimental.pallas import tpu_sc as plsc

N = 10000
E = 320000
D = 128
H = 128
C = 64

NC = 1
NS = 16
NW = NC * NS
NP = 10112


K = 128
_EQ = 32 * NW * K
EP = (E + _EQ - 1) // _EQ * _EQ
NITER = EP // (NW * K)


def _make_segsum(n_in, n_out, w):
    rows = n_out // NS
    assert n_out % NS == 0 and rows % 8 == 0
    nb = 4
    bni = NITER // nb
    assert NITER % nb == 0 and bni % 2 == 0 and bni % 8 == 0

    mesh = plsc.VectorSubcoreMesh(core_axis_name="c", subcore_axis_name="s", num_cores=NC)

    @functools.partial(
        pl.kernel,
        out_type=jax.ShapeDtypeStruct((NC, n_out, w), jnp.float32),
        mesh=mesh,
        scratch_types=[
            pltpu.VMEM((bni, K), jnp.int32),
            pltpu.VMEM((bni, K), jnp.int32),
            pltpu.VMEM((K, w), jnp.float32),
            pltpu.VMEM((K, w), jnp.float32),
            pltpu.VMEM_SHARED((n_out, w), jnp.float32),
            pltpu.SemaphoreType.DMA,
            pltpu.SemaphoreType.DMA,
        ],
    )
    def segsum(h_hbm, src_hbm, dst_hbm, zeros_hbm, out_hbm,
               src_v, dst_v, rows0_v, rows1_v, acc, sem0, sem1):
        c = lax.axis_index("c")
        s = lax.axis_index("s")
        wid = c * NS + s
        row0 = s * rows
        pltpu.sync_copy(zeros_hbm.at[pl.ds(row0, rows)],
                        acc.at[pl.ds(row0, rows)])
        plsc.subcore_barrier()

        for km in range(nb):
            cbase = wid * NITER + km * bni
            pltpu.sync_copy(src_hbm.at[pl.ds(cbase, bni)], src_v)
            pltpu.sync_copy(dst_hbm.at[pl.ds(cbase, bni)], dst_v)

            pltpu.async_copy(h_hbm.at[src_v.at[0]], rows0_v, sem0)

            def pair_body(g, carry):
                j = 2 * g
                pltpu.async_copy(h_hbm.at[src_v.at[j + 1]], rows1_v, sem1)
                pltpu.make_async_copy(h_hbm.at[src_v.at[j]], rows0_v,
                                      sem0).wait()
                pltpu.sync_copy(rows0_v, acc.at[dst_v.at[j]], add=True)
                pltpu.async_copy(h_hbm.at[src_v.at[j + 2]], rows0_v, sem0)
                pltpu.make_async_copy(h_hbm.at[src_v.at[j + 1]], rows1_v,
                                      sem1).wait()
                pltpu.sync_copy(rows1_v, acc.at[dst_v.at[j + 1]], add=True)
                return carry

            lax.fori_loop(0, bni // 2 - 1, pair_body, 0)
            j = bni - 2
            pltpu.async_copy(h_hbm.at[src_v.at[j + 1]], rows1_v, sem1)
            pltpu.make_async_copy(h_hbm.at[src_v.at[j]], rows0_v, sem0).wait()
            pltpu.sync_copy(rows0_v, acc.at[dst_v.at[j]], add=True)
            pltpu.make_async_copy(h_hbm.at[src_v.at[j + 1]], rows1_v,
                                  sem1).wait()
            pltpu.sync_copy(rows1_v, acc.at[dst_v.at[j + 1]], add=True)

        plsc.subcore_barrier()
        pltpu.sync_copy(acc.at[pl.ds(row0, rows)],
                        out_hbm.at[c, pl.ds(row0, rows)])

    return segsum


_segsum_feat = _make_segsum(N, NP, H)
_segsum_hid = _make_segsum(NP, NP, H)


def _make_linear(n, din, dout, bn, relu):
    def body(p_ref, wt_ref, b_ref, o_ref):
        x = p_ref[0]
        y = jnp.dot(x, wt_ref[...], preferred_element_type=jnp.float32)
        y = y + b_ref[...]
        o_ref[...] = jnp.maximum(y, 0.0) if relu else y

    return pl.pallas_call(
        body,
        grid=(n // bn,),
        in_specs=[
            pl.BlockSpec((NC, bn, din), lambda i: (0, i, 0)),
            pl.BlockSpec((din, dout), lambda i: (0, 0)),
            pl.BlockSpec((1, dout), lambda i: (0, 0)),
        ],
        out_specs=pl.BlockSpec((bn, dout), lambda i: (i, 0)),
        out_shape=jax.ShapeDtypeStruct((n, dout), jnp.float32),
    )


def _make_final(n, din, dc, bn):
    def body(p_ref, wt_ref, b_ref, o_ref):
        x = p_ref[0]
        z = jnp.dot(x, wt_ref[...], preferred_element_type=jnp.float32)
        z = z + b_ref[...]
        m = jnp.max(z, axis=1, keepdims=True)
        ez = z - m
        lse = jnp.log(jnp.sum(jnp.exp(ez), axis=1, keepdims=True))
        o_ref[...] = ez - lse

    return pl.pallas_call(
        body,
        grid=(n // bn,),
        in_specs=[
            pl.BlockSpec((NC, bn, din), lambda i: (0, i, 0)),
            pl.BlockSpec((din, dc), lambda i: (0, 0)),
            pl.BlockSpec((1, dc), lambda i: (0, 0)),
        ],
        out_specs=pl.BlockSpec((bn, dc), lambda i: (i, 0)),
        out_shape=jax.ShapeDtypeStruct((n, dc), jnp.float32),
    )


_BN = 632
_linear0 = _make_linear(NP, D, H, _BN, True)
_linear1 = _make_linear(NP, H, H, _BN, True)
_final = _make_final(NP, H, C, _BN)


def kernel(features, labels, mask, edge_index, W0, b0, W1, b1, W2, b2):
    pad = EP - E
    src = jnp.concatenate(
        [edge_index[0], jnp.zeros((pad,), jnp.int32)]).reshape(EP // K, K)
    pad_dst = N + (jnp.arange(pad, dtype=jnp.int32) % (NP - N))
    dst = jnp.concatenate([edge_index[1], pad_dst]).reshape(EP // K, K)
    zeros128 = jnp.zeros((NP, H), jnp.float32)
    w0t = W0.T
    w1t = W1.T
    w2t = W2.T
    b0r = b0.reshape(1, H)
    b1r = b1.reshape(1, H)
    b2r = b2.reshape(1, C)

    p0 = _segsum_feat(features, src, dst, zeros128)
    h0 = _linear0(p0, w0t, b0r)
    p1 = _segsum_hid(h0, src, dst, zeros128)
    h1 = _linear1(p1, w1t, b1r)
    p2 = _segsum_hid(h1, src, dst, zeros128)
    out = _final(p2, w2t, b2r)
    return out[:N]


# speedup vs baseline: 2.6486x; 2.6486x over previous
import functools

import jax
import jax.numpy as jnp
from jax import lax
from jax.experimental import pallas as pl
from jax.experimental.pallas import tpu as pltpu
from jax.experimental.pallas import tpu_sc as plsc

N = 10000
E = 320000
D = 128
H = 128
C = 64

NC = 2
NS = 16
NW = NC * NS
NP = 10112

K = 80
EPT = E // NW
NITER = EPT // K
assert E % NW == 0 and EPT % K == 0 and K % 8 == 0 and NITER % 2 == 1


def _make_segsum(n_in, n_out, w):
    rows = n_out // NS
    assert n_out % NS == 0 and rows % 8 == 0

    mesh = plsc.VectorSubcoreMesh(core_axis_name="c", subcore_axis_name="s",
                                  num_cores=NC)

    @functools.partial(
        pl.kernel,
        out_type=jax.ShapeDtypeStruct((NC, n_out, w), jnp.float32),
        mesh=mesh,
        scratch_types=[
            pltpu.VMEM((K,), jnp.int32),
            pltpu.VMEM((K,), jnp.int32),
            pltpu.VMEM((K,), jnp.int32),
            pltpu.VMEM((K,), jnp.int32),
            pltpu.VMEM((K, w), jnp.float32),
            pltpu.VMEM((K, w), jnp.float32),
            pltpu.VMEM_SHARED((n_out, w), jnp.float32),
            pltpu.SemaphoreType.DMA,
            pltpu.SemaphoreType.DMA,
        ],
    )
    def segsum(h_hbm, src_hbm, dst_hbm, zeros_hbm, out_hbm,
               src_a, dst_a, src_b, dst_b, rows_a, rows_b, acc,
               sem_a, sem_b):
        c = lax.axis_index("c")
        s = lax.axis_index("s")
        wid = c * NS + s
        row0 = s * rows
        pltpu.sync_copy(zeros_hbm.at[pl.ds(row0, rows)],
                        acc.at[pl.ds(row0, rows)])
        plsc.subcore_barrier()

        ebase = wid * EPT

        def fetch_idx(j, sv, dv):
            b = ebase + j * K
            pltpu.sync_copy(src_hbm.at[pl.ds(b, K)], sv)
            pltpu.sync_copy(dst_hbm.at[pl.ds(b, K)], dv)

        fetch_idx(0, src_a, dst_a)
        pltpu.async_copy(h_hbm.at[src_a], rows_a, sem_a)

        def pair_body(g, carry):
            j = 2 * g
            fetch_idx(j + 1, src_b, dst_b)
            pltpu.async_copy(h_hbm.at[src_b], rows_b, sem_b)
            pltpu.make_async_copy(h_hbm.at[src_a], rows_a, sem_a).wait()
            pltpu.sync_copy(rows_a, acc.at[dst_a], add=True)
            fetch_idx(j + 2, src_a, dst_a)
            pltpu.async_copy(h_hbm.at[src_a], rows_a, sem_a)
            pltpu.make_async_copy(h_hbm.at[src_b], rows_b, sem_b).wait()
            pltpu.sync_copy(rows_b, acc.at[dst_b], add=True)
            return carry

        lax.fori_loop(0, NITER // 2, pair_body, 0)
        pltpu.make_async_copy(h_hbm.at[src_a], rows_a, sem_a).wait()
        pltpu.sync_copy(rows_a, acc.at[dst_a], add=True)

        plsc.subcore_barrier()
        pltpu.sync_copy(acc.at[pl.ds(row0, rows)],
                        out_hbm.at[c, pl.ds(row0, rows)])

    return segsum


_segsum_feat = _make_segsum(N, NP, H)
_segsum_hid = _make_segsum(NP, NP, H)


def _make_linear(n, din, dout, bn, relu):
    def body(p_ref, wt_ref, b_ref, o_ref):
        x = p_ref[0] + p_ref[1]
        y = jnp.dot(x, wt_ref[...], preferred_element_type=jnp.float32)
        y = y + b_ref[...]
        o_ref[...] = jnp.maximum(y, 0.0) if relu else y

    return pl.pallas_call(
        body,
        grid=(n // bn,),
        in_specs=[
            pl.BlockSpec((NC, bn, din), lambda i: (0, i, 0)),
            pl.BlockSpec((din, dout), lambda i: (0, 0)),
            pl.BlockSpec((1, dout), lambda i: (0, 0)),
        ],
        out_specs=pl.BlockSpec((bn, dout), lambda i: (i, 0)),
        out_shape=jax.ShapeDtypeStruct((n, dout), jnp.float32),
    )


def _make_final(n, din, dc, bn):
    def body(p_ref, wt_ref, b_ref, o_ref):
        x = p_ref[0] + p_ref[1]
        z = jnp.dot(x, wt_ref[...], preferred_element_type=jnp.float32)
        z = z + b_ref[...]
        m = jnp.max(z, axis=1, keepdims=True)
        ez = z - m
        lse = jnp.log(jnp.sum(jnp.exp(ez), axis=1, keepdims=True))
        o_ref[...] = ez - lse

    return pl.pallas_call(
        body,
        grid=(n // bn,),
        in_specs=[
            pl.BlockSpec((NC, bn, din), lambda i: (0, i, 0)),
            pl.BlockSpec((din, dc), lambda i: (0, 0)),
            pl.BlockSpec((1, dc), lambda i: (0, 0)),
        ],
        out_specs=pl.BlockSpec((bn, dc), lambda i: (i, 0)),
        out_shape=jax.ShapeDtypeStruct((n, dc), jnp.float32),
    )


_BN = 632
_linear0 = _make_linear(NP, D, H, _BN, True)
_linear1 = _make_linear(NP, H, H, _BN, True)
_final = _make_final(NP, H, C, _BN)


def kernel(features, labels, mask, edge_index, W0, b0, W1, b1, W2, b2):
    src = edge_index[0]
    dst = edge_index[1]
    zeros128 = jnp.zeros((NP, H), jnp.float32)
    w0t = W0.T
    w1t = W1.T
    w2t = W2.T
    b0r = b0.reshape(1, H)
    b1r = b1.reshape(1, H)
    b2r = b2.reshape(1, C)

    p0 = _segsum_feat(features, src, dst, zeros128)
    h0 = _linear0(p0, w0t, b0r)
    p1 = _segsum_hid(h0, src, dst, zeros128)
    h1 = _linear1(p1, w1t, b1r)
    p2 = _segsum_hid(h1, src, dst, zeros128)
    out = _final(p2, w2t, b2r)
    return out[:N]


# speedup vs baseline: 4.5143x; 1.7044x over previous
import functools

import jax
import jax.numpy as jnp
from jax import lax
from jax.experimental import pallas as pl
from jax.experimental.pallas import tpu as pltpu
from jax.experimental.pallas import tpu_sc as plsc

N = 10000
E = 320000
D = 128
H = 128
C = 64

NC = 2
NS = 16
NW = NC * NS
NP = 10112

K = 80
EPT = E // NW
NITER = EPT // K
assert E % NW == 0 and EPT % K == 0 and K % 8 == 0 and NITER % 2 == 1


def _make_segsum(n_in, n_out, w):
    rows = n_out // NS
    assert n_out % NS == 0 and rows % 8 == 0

    mesh = plsc.VectorSubcoreMesh(core_axis_name="c", subcore_axis_name="s",
                                  num_cores=NC)

    nslot = 4

    @functools.partial(
        pl.kernel,
        out_type=jax.ShapeDtypeStruct((NC, n_out, w), jnp.float32),
        mesh=mesh,
        scratch_types=[
            [pltpu.VMEM((K,), jnp.int32)] * nslot,
            [pltpu.VMEM((K,), jnp.int32)] * nslot,
            [pltpu.VMEM((K, w), jnp.float32)] * nslot,
            pltpu.VMEM_SHARED((n_out, w), jnp.float32),
            [pltpu.SemaphoreType.DMA] * nslot,
            [pltpu.SemaphoreType.DMA] * nslot,
        ],
    )
    def segsum(h_hbm, src_hbm, dst_hbm, zeros_hbm, out_hbm,
               src_v, dst_v, rows_v, acc, gsem, isem):
        c = lax.axis_index("c")
        s = lax.axis_index("s")
        wid = c * NS + s
        row0 = s * rows
        pltpu.sync_copy(zeros_hbm.at[pl.ds(row0, rows)],
                        acc.at[pl.ds(row0, rows)])
        plsc.subcore_barrier()

        ebase = wid * EPT

        for j in range(nslot - 1):
            b = ebase + j * K
            pltpu.sync_copy(src_hbm.at[pl.ds(b, K)], src_v[j])
            pltpu.sync_copy(dst_hbm.at[pl.ds(b, K)], dst_v[j])
            pltpu.async_copy(h_hbm.at[src_v[j]], rows_v[j], gsem[j])

        def quad_body(g, carry):
            for i in range(nslot):
                j = nslot * g + i
                p = i
                p3 = (i + nslot - 1) % nslot
                jn = j + nslot - 1

                @pl.when(jn < NITER)
                def _():
                    b = ebase + jn * K
                    pltpu.async_copy(src_hbm.at[pl.ds(b, K)], src_v[p3],
                                     isem[p3])
                    pltpu.async_copy(dst_hbm.at[pl.ds(b, K)], dst_v[p3],
                                     isem[p3])

                @pl.when(j < NITER)
                def _():
                    pltpu.make_async_copy(h_hbm.at[src_v[p]], rows_v[p],
                                          gsem[p]).wait()
                    pltpu.sync_copy(rows_v[p], acc.at[dst_v[p]], add=True)

                @pl.when(jn < NITER)
                def _():
                    b = ebase + jn * K
                    pltpu.make_async_copy(src_hbm.at[pl.ds(b, K)], src_v[p3],
                                          isem[p3]).wait()
                    pltpu.make_async_copy(dst_hbm.at[pl.ds(b, K)], dst_v[p3],
                                          isem[p3]).wait()
                    pltpu.async_copy(h_hbm.at[src_v[p3]], rows_v[p3],
                                     gsem[p3])
            return carry

        lax.fori_loop(0, (NITER + nslot - 1) // nslot, quad_body, 0)

        plsc.subcore_barrier()
        pltpu.sync_copy(acc.at[pl.ds(row0, rows)],
                        out_hbm.at[c, pl.ds(row0, rows)])

    return segsum


_segsum_feat = _make_segsum(N, NP, H)
_segsum_hid = _make_segsum(NP, NP, H)


def _make_linear(n, din, dout, bn, relu):
    def body(p_ref, wt_ref, b_ref, o_ref):
        x = p_ref[0] + p_ref[1]
        y = jnp.dot(x, wt_ref[...], preferred_element_type=jnp.float32)
        y = y + b_ref[...]
        o_ref[...] = jnp.maximum(y, 0.0) if relu else y

    return pl.pallas_call(
        body,
        grid=(n // bn,),
        in_specs=[
            pl.BlockSpec((NC, bn, din), lambda i: (0, i, 0)),
            pl.BlockSpec((din, dout), lambda i: (0, 0)),
            pl.BlockSpec((1, dout), lambda i: (0, 0)),
        ],
        out_specs=pl.BlockSpec((bn, dout), lambda i: (i, 0)),
        out_shape=jax.ShapeDtypeStruct((n, dout), jnp.float32),
    )


def _make_final(n, din, dc, bn):
    def body(p_ref, wt_ref, b_ref, o_ref):
        x = p_ref[0] + p_ref[1]
        z = jnp.dot(x, wt_ref[...], preferred_element_type=jnp.float32)
        z = z + b_ref[...]
        m = jnp.max(z, axis=1, keepdims=True)
        ez = z - m
        lse = jnp.log(jnp.sum(jnp.exp(ez), axis=1, keepdims=True))
        o_ref[...] = ez - lse

    return pl.pallas_call(
        body,
        grid=(n // bn,),
        in_specs=[
            pl.BlockSpec((NC, bn, din), lambda i: (0, i, 0)),
            pl.BlockSpec((din, dc), lambda i: (0, 0)),
            pl.BlockSpec((1, dc), lambda i: (0, 0)),
        ],
        out_specs=pl.BlockSpec((bn, dc), lambda i: (i, 0)),
        out_shape=jax.ShapeDtypeStruct((n, dc), jnp.float32),
    )


_BN = 632
_linear0 = _make_linear(NP, D, H, _BN, True)
_linear1 = _make_linear(NP, H, H, _BN, True)
_final = _make_final(NP, H, C, _BN)


def kernel(features, labels, mask, edge_index, W0, b0, W1, b1, W2, b2):
    src = edge_index[0]
    dst = edge_index[1]
    zeros128 = jnp.zeros((NP, H), jnp.float32)
    w0t = W0.T
    w1t = W1.T
    w2t = W2.T
    b0r = b0.reshape(1, H)
    b1r = b1.reshape(1, H)
    b2r = b2.reshape(1, C)

    p0 = _segsum_feat(features, src, dst, zeros128)
    h0 = _linear0(p0, w0t, b0r)
    p1 = _segsum_hid(h0, src, dst, zeros128)
    h1 = _linear1(p1, w1t, b1r)
    p2 = _segsum_hid(h1, src, dst, zeros128)
    out = _final(p2, w2t, b2r)
    return out[:N]


# speedup vs baseline: 4.5820x; 1.0150x over previous
import functools

import jax
import jax.numpy as jnp
from jax import lax
from jax.experimental import pallas as pl
from jax.experimental.pallas import tpu as pltpu
from jax.experimental.pallas import tpu_sc as plsc

N = 10000
E = 320000
D = 128
H = 128
C = 64

NC = 2
NS = 16
NW = NC * NS
NP = 10112

K = 80
EPT = E // NW
NITER = EPT // K
assert E % NW == 0 and EPT % K == 0 and K % 8 == 0 and NITER % 2 == 1


def _make_segsum(n_in, n_out, w, tc_tiling=True):
    rows = n_out // NS
    assert n_out % NS == 0 and rows % 8 == 0

    mesh = plsc.VectorSubcoreMesh(core_axis_name="c", subcore_axis_name="s",
                                  num_cores=NC)

    nslot = 4

    @functools.partial(
        pl.kernel,
        out_type=jax.ShapeDtypeStruct((NC, n_out, w), jnp.float32),
        mesh=mesh,
        compiler_params=pltpu.CompilerParams(use_tc_tiling_on_sc=tc_tiling),
        scratch_types=[
            [pltpu.VMEM((K,), jnp.int32)] * nslot,
            [pltpu.VMEM((K,), jnp.int32)] * nslot,
            [pltpu.VMEM((K, w), jnp.float32)] * nslot,
            pltpu.VMEM_SHARED((n_out, w), jnp.float32),
            [pltpu.SemaphoreType.DMA] * nslot,
            [pltpu.SemaphoreType.DMA] * nslot,
        ],
    )
    def segsum(h_hbm, src_hbm, dst_hbm, zeros_hbm, out_hbm,
               src_v, dst_v, rows_v, acc, gsem, isem):
        c = lax.axis_index("c")
        s = lax.axis_index("s")
        wid = c * NS + s
        row0 = s * rows
        pltpu.sync_copy(zeros_hbm.at[pl.ds(row0, rows)],
                        acc.at[pl.ds(row0, rows)])
        plsc.subcore_barrier()

        ebase = wid * EPT

        for j in range(nslot - 1):
            b = ebase + j * K
            pltpu.sync_copy(src_hbm.at[pl.ds(b, K)], src_v[j])
            pltpu.sync_copy(dst_hbm.at[pl.ds(b, K)], dst_v[j])
            pltpu.async_copy(h_hbm.at[src_v[j]], rows_v[j], gsem[j])

        def quad_body(g, carry):
            for i in range(nslot):
                j = nslot * g + i
                p = i
                p3 = (i + nslot - 1) % nslot
                jn = j + nslot - 1

                @pl.when(jn < NITER)
                def _():
                    b = ebase + jn * K
                    pltpu.async_copy(src_hbm.at[pl.ds(b, K)], src_v[p3],
                                     isem[p3])
                    pltpu.async_copy(dst_hbm.at[pl.ds(b, K)], dst_v[p3],
                                     isem[p3])

                @pl.when(j < NITER)
                def _():
                    pltpu.make_async_copy(h_hbm.at[src_v[p]], rows_v[p],
                                          gsem[p]).wait()
                    pltpu.sync_copy(rows_v[p], acc.at[dst_v[p]], add=True)

                @pl.when(jn < NITER)
                def _():
                    b = ebase + jn * K
                    pltpu.make_async_copy(src_hbm.at[pl.ds(b, K)], src_v[p3],
                                          isem[p3]).wait()
                    pltpu.make_async_copy(dst_hbm.at[pl.ds(b, K)], dst_v[p3],
                                          isem[p3]).wait()
                    pltpu.async_copy(h_hbm.at[src_v[p3]], rows_v[p3],
                                     gsem[p3])
            return carry

        lax.fori_loop(0, (NITER + nslot - 1) // nslot, quad_body, 0)

        plsc.subcore_barrier()
        pltpu.sync_copy(acc.at[pl.ds(row0, rows)],
                        out_hbm.at[c, pl.ds(row0, rows)])

    return segsum


_segsum_feat = _make_segsum(N, NP, H)
_segsum_hid = _make_segsum(NP, NP, H)
_segsum_y2 = _make_segsum(NP, NP, C, tc_tiling=False)


def _make_linear(n, din, dout, bn, relu):
    def body(p_ref, wt_ref, b_ref, o_ref):
        x = p_ref[0] + p_ref[1]
        y = jnp.dot(x, wt_ref[...], preferred_element_type=jnp.float32)
        y = y + b_ref[...]
        o_ref[...] = jnp.maximum(y, 0.0) if relu else y

    return pl.pallas_call(
        body,
        grid=(n // bn,),
        in_specs=[
            pl.BlockSpec((NC, bn, din), lambda i: (0, i, 0)),
            pl.BlockSpec((din, dout), lambda i: (0, 0)),
            pl.BlockSpec((1, dout), lambda i: (0, 0)),
        ],
        out_specs=pl.BlockSpec((bn, dout), lambda i: (i, 0)),
        out_shape=jax.ShapeDtypeStruct((n, dout), jnp.float32),
    )


def _make_linear_fused2(n, din, dh, dc, bn):
    def body(p_ref, w1t_ref, b1_ref, w2t_ref, h_ref, y_ref):
        x = p_ref[0] + p_ref[1]
        h = jnp.dot(x, w1t_ref[...], preferred_element_type=jnp.float32)
        h = jnp.maximum(h + b1_ref[...], 0.0)
        h_ref[...] = h
        y_ref[...] = jnp.dot(h, w2t_ref[...],
                             preferred_element_type=jnp.float32)

    return pl.pallas_call(
        body,
        grid=(n // bn,),
        in_specs=[
            pl.BlockSpec((NC, bn, din), lambda i: (0, i, 0)),
            pl.BlockSpec((din, dh), lambda i: (0, 0)),
            pl.BlockSpec((1, dh), lambda i: (0, 0)),
            pl.BlockSpec((dh, dc), lambda i: (0, 0)),
        ],
        out_specs=[
            pl.BlockSpec((bn, dh), lambda i: (i, 0)),
            pl.BlockSpec((bn, dc), lambda i: (i, 0)),
        ],
        out_shape=[
            jax.ShapeDtypeStruct((n, dh), jnp.float32),
            jax.ShapeDtypeStruct((n, dc), jnp.float32),
        ],
    )


def _make_final(n, dc, bn):
    def body(p_ref, b_ref, o_ref):
        z = p_ref[0] + p_ref[1] + b_ref[...]
        m = jnp.max(z, axis=1, keepdims=True)
        ez = z - m
        lse = jnp.log(jnp.sum(jnp.exp(ez), axis=1, keepdims=True))
        o_ref[...] = ez - lse

    return pl.pallas_call(
        body,
        grid=(n // bn,),
        in_specs=[
            pl.BlockSpec((NC, bn, dc), lambda i: (0, i, 0)),
            pl.BlockSpec((1, dc), lambda i: (0, 0)),
        ],
        out_specs=pl.BlockSpec((bn, dc), lambda i: (i, 0)),
        out_shape=jax.ShapeDtypeStruct((n, dc), jnp.float32),
    )


_BN = 632
_linear0 = _make_linear(NP, D, H, _BN, True)
_linear1f = _make_linear_fused2(NP, H, H, C, _BN)
_final = _make_final(NP, C, _BN)


def kernel(features, labels, mask, edge_index, W0, b0, W1, b1, W2, b2):
    src = edge_index[0]
    dst = edge_index[1]
    zeros128 = jnp.zeros((NP, H), jnp.float32)
    zeros64 = jnp.zeros((NP, C), jnp.float32)
    w0t = W0.T
    w1t = W1.T
    w2t = W2.T
    b0r = b0.reshape(1, H)
    b1r = b1.reshape(1, H)
    b2r = b2.reshape(1, C)

    p0 = _segsum_feat(features, src, dst, zeros128)
    h0 = _linear0(p0, w0t, b0r)
    p1 = _segsum_hid(h0, src, dst, zeros128)
    h1, y2 = _linear1f(p1, w1t, b1r, w2t)
    p2 = _segsum_y2(y2, src, dst, zeros64)
    out = _final(p2, b2r)
    return out[:N]


# speedup vs baseline: 4.6556x; 1.0160x over previous
import functools

import jax
import jax.numpy as jnp
from jax import lax
from jax.experimental import pallas as pl
from jax.experimental.pallas import tpu as pltpu
from jax.experimental.pallas import tpu_sc as plsc

N = 10000
E = 320000
D = 128
H = 128
C = 64

NC = 2
NS = 16
NW = NC * NS
NP = 10112

K = 80
EPT = E // NW
NITER = EPT // K
assert E % NW == 0 and EPT % K == 0 and K % 8 == 0 and NITER % 2 == 1


def _make_segsum(n_in, n_out, w, tc_tiling=False):
    rows = n_out // NS
    assert n_out % NS == 0 and rows % 8 == 0

    mesh = plsc.VectorSubcoreMesh(core_axis_name="c", subcore_axis_name="s",
                                  num_cores=NC)

    nslot = 4

    @functools.partial(
        pl.kernel,
        out_type=jax.ShapeDtypeStruct((NC, n_out, w), jnp.float32),
        mesh=mesh,
        compiler_params=pltpu.CompilerParams(use_tc_tiling_on_sc=tc_tiling),
        scratch_types=[
            [pltpu.VMEM((K,), jnp.int32)] * nslot,
            [pltpu.VMEM((K,), jnp.int32)] * nslot,
            [pltpu.VMEM((K, w), jnp.float32)] * nslot,
            pltpu.VMEM_SHARED((n_out, w), jnp.float32),
            [pltpu.SemaphoreType.DMA] * nslot,
            [pltpu.SemaphoreType.DMA] * nslot,
        ],
    )
    def segsum(h_hbm, ei_hbm, zeros_hbm, out_hbm,
               src_v, dst_v, rows_v, acc, gsem, isem):
        c = lax.axis_index("c")
        s = lax.axis_index("s")
        wid = c * NS + s
        row0 = s * rows
        pltpu.sync_copy(zeros_hbm.at[pl.ds(row0, rows)],
                        acc.at[pl.ds(row0, rows)])
        plsc.subcore_barrier()

        ebase = wid * EPT

        for j in range(nslot - 1):
            b = ebase + j * K
            pltpu.sync_copy(ei_hbm.at[0, pl.ds(b, K)], src_v[j])
            pltpu.sync_copy(ei_hbm.at[1, pl.ds(b, K)], dst_v[j])
            pltpu.async_copy(h_hbm.at[src_v[j]], rows_v[j], gsem[j])

        def quad_body(g, carry):
            for i in range(nslot):
                j = nslot * g + i
                p = i
                p3 = (i + nslot - 1) % nslot
                jn = j + nslot - 1

                @pl.when(jn < NITER)
                def _():
                    b = ebase + jn * K
                    pltpu.async_copy(ei_hbm.at[0, pl.ds(b, K)], src_v[p3],
                                     isem[p3])
                    pltpu.async_copy(ei_hbm.at[1, pl.ds(b, K)], dst_v[p3],
                                     isem[p3])

                @pl.when(j < NITER)
                def _():
                    pltpu.make_async_copy(h_hbm.at[src_v[p]], rows_v[p],
                                          gsem[p]).wait()
                    pltpu.sync_copy(rows_v[p], acc.at[dst_v[p]], add=True)

                @pl.when(jn < NITER)
                def _():
                    b = ebase + jn * K
                    pltpu.make_async_copy(ei_hbm.at[0, pl.ds(b, K)], src_v[p3],
                                          isem[p3]).wait()
                    pltpu.make_async_copy(ei_hbm.at[1, pl.ds(b, K)], dst_v[p3],
                                          isem[p3]).wait()
                    pltpu.async_copy(h_hbm.at[src_v[p3]], rows_v[p3],
                                     gsem[p3])
            return carry

        lax.fori_loop(0, (NITER + nslot - 1) // nslot, quad_body, 0)

        plsc.subcore_barrier()
        pltpu.sync_copy(acc.at[pl.ds(row0, rows)],
                        out_hbm.at[c, pl.ds(row0, rows)])

    return segsum


_segsum_feat = _make_segsum(N, NP, H)
_segsum_hid = _make_segsum(NP, NP, H)
_segsum_y2 = _make_segsum(NP, NP, C, tc_tiling=False)


def _make_linear(n, din, dout, bn, relu):
    def body(p_ref, wt_ref, b_ref, o_ref):
        x = p_ref[0] + p_ref[1]
        y = jnp.dot(x, wt_ref[...], preferred_element_type=jnp.float32)
        y = y + b_ref[...]
        o_ref[...] = jnp.maximum(y, 0.0) if relu else y

    return pl.pallas_call(
        body,
        grid=(n // bn,),
        in_specs=[
            pl.BlockSpec((NC, bn, din), lambda i: (0, i, 0)),
            pl.BlockSpec((din, dout), lambda i: (0, 0)),
            pl.BlockSpec((1, dout), lambda i: (0, 0)),
        ],
        out_specs=pl.BlockSpec((bn, dout), lambda i: (i, 0)),
        out_shape=jax.ShapeDtypeStruct((n, dout), jnp.float32),
    )


def _make_linear_fused2(n, din, dh, dc, bn):
    def body(p_ref, w1t_ref, b1_ref, w2t_ref, h_ref, y_ref):
        x = p_ref[0] + p_ref[1]
        h = jnp.dot(x, w1t_ref[...], preferred_element_type=jnp.float32)
        h = jnp.maximum(h + b1_ref[...], 0.0)
        h_ref[...] = h
        y_ref[...] = jnp.dot(h, w2t_ref[...],
                             preferred_element_type=jnp.float32)

    return pl.pallas_call(
        body,
        grid=(n // bn,),
        in_specs=[
            pl.BlockSpec((NC, bn, din), lambda i: (0, i, 0)),
            pl.BlockSpec((din, dh), lambda i: (0, 0)),
            pl.BlockSpec((1, dh), lambda i: (0, 0)),
            pl.BlockSpec((dh, dc), lambda i: (0, 0)),
        ],
        out_specs=[
            pl.BlockSpec((bn, dh), lambda i: (i, 0)),
            pl.BlockSpec((bn, dc), lambda i: (i, 0)),
        ],
        out_shape=[
            jax.ShapeDtypeStruct((n, dh), jnp.float32),
            jax.ShapeDtypeStruct((n, dc), jnp.float32),
        ],
    )


def _make_final(n, dc, bn):
    def body(p_ref, b_ref, o_ref):
        z = p_ref[0] + p_ref[1] + b_ref[...]
        m = jnp.max(z, axis=1, keepdims=True)
        ez = z - m
        lse = jnp.log(jnp.sum(jnp.exp(ez), axis=1, keepdims=True))
        o_ref[...] = ez - lse

    return pl.pallas_call(
        body,
        grid=(n // bn,),
        in_specs=[
            pl.BlockSpec((NC, bn, dc), lambda i: (0, i, 0)),
            pl.BlockSpec((1, dc), lambda i: (0, 0)),
        ],
        out_specs=pl.BlockSpec((bn, dc), lambda i: (i, 0)),
        out_shape=jax.ShapeDtypeStruct((n, dc), jnp.float32),
    )


_BN = 632
_linear0 = _make_linear(NP, D, H, _BN, True)
_linear1f = _make_linear_fused2(NP, H, H, C, _BN)
_final = _make_final(N, C, 400)


def kernel(features, labels, mask, edge_index, W0, b0, W1, b1, W2, b2):
    zeros128 = jnp.zeros((NP, H), jnp.float32)
    zeros64 = jnp.zeros((NP, C), jnp.float32)
    w0t = W0.T
    w1t = W1.T
    w2t = W2.T
    b0r = b0.reshape(1, H)
    b1r = b1.reshape(1, H)
    b2r = b2.reshape(1, C)

    p0 = _segsum_feat(features, edge_index, zeros128)
    h0 = _linear0(p0, w0t, b0r)
    p1 = _segsum_hid(h0, edge_index, zeros128)
    h1, y2 = _linear1f(p1, w1t, b1r, w2t)
    p2 = _segsum_y2(y2, edge_index, zeros64)
    out = _final(p2, b2r)
    return out
